# baseline (speedup 1.0000x reference)
import functools

import jax
import jax.numpy as jnp
from jax import lax
from jax.experimental import pallas as pl
from jax.experimental.pallas import tpu as pltpu
from jax.experimental.pallas import tpu_sc as plsc

NUM_EMB = 1000000
DIM = 32
B = 16384

_NC = 2
_NS = 16
_NW = _NC * _NS
_BPW = B // _NW

_mesh = plsc.VectorSubcoreMesh(core_axis_name="c", subcore_axis_name="s")


@functools.partial(
    pl.kernel,
    mesh=_mesh,
    compiler_params=pltpu.CompilerParams(use_tc_tiling_on_sc=False),
    out_type=jax.ShapeDtypeStruct((2, B, DIM), jnp.bfloat16),
    scratch_types=[
        pltpu.VMEM((_BPW,), jnp.int32),
        pltpu.VMEM((_BPW, DIM), jnp.bfloat16),
        pltpu.VMEM((_BPW, DIM), jnp.bfloat16),
        pltpu.SemaphoreType.DMA,
    ],
)
def _emb_lookup(idx_hbm, wr_hbm, wi_hbm, out_hbm, idx_v, rows_r, rows_i, sem):
    wid = lax.axis_index("s") * _NC + lax.axis_index("c")
    base = wid * _BPW
    pltpu.sync_copy(idx_hbm.at[wid], idx_v)

    def grp_body(g, _):
        grp = idx_v[pl.ds(g * 16, 16)]
        for lane in range(16):
            row = grp[lane]
            i = g * 16 + lane
            pltpu.async_copy(
                wr_hbm.at[pl.ds(row, 1), :], rows_r.at[pl.ds(i, 1), :], sem)
            pltpu.async_copy(
                wi_hbm.at[pl.ds(row, 1), :], rows_i.at[pl.ds(i, 1), :], sem)
        return ()

    lax.fori_loop(0, _BPW // 16, grp_body, ())
    pltpu.make_async_copy(wr_hbm.at[pl.ds(0, _BPW), :], rows_r, sem).wait()
    pltpu.make_async_copy(wi_hbm.at[pl.ds(0, _BPW), :], rows_i, sem).wait()
    pltpu.sync_copy(rows_r, out_hbm.at[0, pl.ds(base, _BPW)])
    pltpu.sync_copy(rows_i, out_hbm.at[1, pl.ds(base, _BPW)])


@jax.jit
def kernel(index, W_real, W_img):
    idx = index.astype(jnp.int32).reshape(_NW, _BPW)
    out16 = _emb_lookup(
        idx, W_real.astype(jnp.bfloat16), W_img.astype(jnp.bfloat16))
    return out16.astype(jnp.float32)


# speedup vs baseline: 1.1676x; 1.1676x over previous
import functools

import jax
import jax.numpy as jnp
from jax import lax
from jax.experimental import pallas as pl
from jax.experimental.pallas import tpu as pltpu
from jax.experimental.pallas import tpu_sc as plsc

NUM_EMB = 1000000
DIM = 32
B = 16384

_NC = 2
_NS = 16
_NW = _NC * _NS
_BPW = B // _NW
_HALF = _BPW // 2

_mesh = plsc.VectorSubcoreMesh(core_axis_name="c", subcore_axis_name="s")


@functools.partial(
    pl.kernel,
    mesh=_mesh,
    out_type=jax.ShapeDtypeStruct((2 * B * DIM,), jnp.float32),
    scratch_types=[
        pltpu.VMEM((_BPW,), jnp.int32),
        pltpu.VMEM((_HALF * DIM,), jnp.float32),
        pltpu.VMEM((_HALF * DIM,), jnp.float32),
        pltpu.SemaphoreType.DMA,
    ],
)
def _emb_lookup(idx_hbm, wr_hbm, wi_hbm, out_hbm, idx_v, rows_r, rows_i, sem):
    wid = lax.axis_index("s") * _NC + lax.axis_index("c")
    base = wid * _BPW
    pltpu.sync_copy(idx_hbm.at[wid], idx_v)

    for half in range(2):
        off = half * _HALF

        def grp_body(g, _):
            grp = idx_v[pl.ds(off + g * 16, 16)]
            for lane in range(16):
                start = pl.multiple_of(grp[lane], DIM)
                dst = pl.ds((g * 16 + lane) * DIM, DIM)
                pltpu.async_copy(wr_hbm.at[pl.ds(start, DIM)], rows_r.at[dst], sem)
                pltpu.async_copy(wi_hbm.at[pl.ds(start, DIM)], rows_i.at[dst], sem)
            return ()

        lax.fori_loop(0, _HALF // 16, grp_body, ())
        pltpu.make_async_copy(wr_hbm.at[pl.ds(0, _HALF * DIM)], rows_r, sem).wait()
        pltpu.make_async_copy(wi_hbm.at[pl.ds(0, _HALF * DIM)], rows_i, sem).wait()
        pltpu.sync_copy(
            rows_r, out_hbm.at[pl.ds((base + off) * DIM, _HALF * DIM)])
        pltpu.sync_copy(
            rows_i, out_hbm.at[pl.ds(B * DIM + (base + off) * DIM, _HALF * DIM)])


@jax.jit
def kernel(index, W_real, W_img):
    idx = (index.astype(jnp.int32) * DIM).reshape(_NW, _BPW)
    out = _emb_lookup(idx, W_real.reshape(-1), W_img.reshape(-1))
    return out.reshape(2, B, DIM)


# speedup vs baseline: 1.7546x; 1.5027x over previous
import functools

import jax
import jax.numpy as jnp
from jax import lax
from jax.experimental import pallas as pl
from jax.experimental.pallas import tpu as pltpu
from jax.experimental.pallas import tpu_sc as plsc

NUM_EMB = 1000000
DIM = 32
B = 16384

_NC = 2
_NS = 16
_NW = _NC * _NS
_BPW = B // _NW
_HALF = _BPW // 2

_mesh = plsc.VectorSubcoreMesh(core_axis_name="c", subcore_axis_name="s")


@functools.partial(
    pl.kernel,
    mesh=_mesh,
    out_type=jax.ShapeDtypeStruct((2, B, DIM), jnp.float32),
    scratch_types=[
        pltpu.VMEM((_BPW,), jnp.int32),
        pltpu.VMEM((_HALF, DIM), jnp.float32),
        pltpu.VMEM((_HALF, DIM), jnp.float32),
        pltpu.SemaphoreType.DMA,
    ],
)
def _emb_lookup(idx_hbm, wr_hbm, wi_hbm, out_hbm, idx_v, rows_r, rows_i, sem):
    wid = lax.axis_index("s") * _NC + lax.axis_index("c")
    base = wid * _BPW
    pltpu.sync_copy(idx_hbm.at[wid], idx_v)

    for half in range(2):
        off = half * _HALF

        def grp_body(g, _):
            grp = idx_v[pl.ds(off + g * 16, 16)]
            for lane in range(16):
                row = grp[lane]
                i = g * 16 + lane
                pltpu.async_copy(
                    wr_hbm.at[pl.ds(row, 1), :], rows_r.at[pl.ds(i, 1), :], sem)
                pltpu.async_copy(
                    wi_hbm.at[pl.ds(row, 1), :], rows_i.at[pl.ds(i, 1), :], sem)
            return ()

        lax.fori_loop(0, _HALF // 16, grp_body, ())
        pltpu.make_async_copy(
            wr_hbm.at[pl.ds(0, _HALF), :], rows_r, sem).wait()
        pltpu.make_async_copy(
            wi_hbm.at[pl.ds(0, _HALF), :], rows_i, sem).wait()
        pltpu.sync_copy(rows_r, out_hbm.at[0, pl.ds(base + off, _HALF)])
        pltpu.sync_copy(rows_i, out_hbm.at[1, pl.ds(base + off, _HALF)])


@jax.jit
def kernel(index, W_real, W_img):
    idx = index.astype(jnp.int32).reshape(_NW, _BPW)
    return _emb_lookup(idx, W_real, W_img)
